# baseline (speedup 1.0000x reference)
import functools

import jax
import jax.numpy as jnp
from jax import lax
from jax.experimental import pallas as pl
from jax.experimental.pallas import tpu as pltpu
from jax.experimental.pallas import tpu_sc as plsc

N = 10000
NPAD = 10240
E = 320000
FEAT = 128
B = 5
P = 2000
NC = 2
NS = 16
NW = NC * NS
EPT = E // NW
CHUNK = 80
CPT = 126
PAD_DST = NPAD - 1
ROWS_PER_TILE = NPAD // NS


def _unpack_chunk(packed, k, sbuf, dbuf):
  for j in range(CHUNK // 16):
    pk = packed[k, pl.ds(j * 16, 16)]
    sbuf[pl.ds(j * 16, 16)] = lax.shift_right_logical(pk, 14)
    dbuf[pl.ds(j * 16, 16)] = jnp.bitwise_and(pk, (1 << 14) - 1)


def _make_sc_scatter(width):
  mesh = plsc.VectorSubcoreMesh(core_axis_name="c", subcore_axis_name="s")

  @functools.partial(
      pl.kernel,
      mesh=mesh,
      out_type=jax.ShapeDtypeStruct((NC, NPAD, width), jnp.float32),
      scratch_types=[
          pltpu.VMEM((CPT, CHUNK), jnp.int32),
          pltpu.VMEM((CHUNK,), jnp.int32),
          pltpu.VMEM((CHUNK,), jnp.int32),
          pltpu.VMEM((CHUNK,), jnp.int32),
          pltpu.VMEM((CHUNK,), jnp.int32),
          pltpu.VMEM((CHUNK,), jnp.int32),
          pltpu.VMEM((CHUNK,), jnp.int32),
          pltpu.VMEM((CHUNK, width), jnp.float32),
          pltpu.VMEM((CHUNK, width), jnp.float32),
          pltpu.VMEM((CHUNK, width), jnp.float32),
          pltpu.VMEM_SHARED((NPAD, width), jnp.float32),
          pltpu.SemaphoreType.DMA,
          pltpu.SemaphoreType.DMA,
          pltpu.SemaphoreType.DMA,
          pltpu.SemaphoreType.DMA,
          pltpu.SemaphoreType.DMA,
          pltpu.SemaphoreType.DMA,
      ],
  )
  def sc_scatter(table, edges, out, pidx, sidx0, sidx1, sidx2,
                 didx0, didx1, didx2, rows0, rows1, rows2, acc,
                 semg0, semg1, semg2, sems0, sems1, sems2):
    c = lax.axis_index("c")
    s = lax.axis_index("s")
    wid = c * NS + s

    def _zero_row(r, _):
      for k in range(width // 16):
        rows0[r, pl.ds(k * 16, 16)] = jnp.zeros((16,), jnp.float32)
      return 0
    lax.fori_loop(0, 64, _zero_row, 0)
    for z in range(ROWS_PER_TILE // 64):
      pltpu.sync_copy(rows0.at[pl.ds(0, 64)],
                      acc.at[pl.ds(s * ROWS_PER_TILE + z * 64, 64)])
    plsc.subcore_barrier()

    pltpu.sync_copy(edges.at[wid], pidx)

    bufs = (rows0, rows1, rows2)
    sbufs = (sidx0, sidx1, sidx2)
    dbufs = (didx0, didx1, didx2)
    gsems = (semg0, semg1, semg2)
    ssems = (sems0, sems1, sems2)

    def _gather(m, r):
      _unpack_chunk(pidx, m, sbufs[r], dbufs[r])
      pltpu.async_copy(table.at[sbufs[r]], bufs[r], gsems[r])

    def _body(m, r, wait_prev, prefetch):
      pltpu.make_async_copy(table.at[sbufs[r]], bufs[r], gsems[r]).wait()
      pltpu.async_copy(bufs[r], acc.at[dbufs[r]], ssems[r], add=True)
      rp = (r + 2) % 3
      if wait_prev:
        pltpu.make_async_copy(bufs[rp], acc.at[dbufs[rp]], ssems[rp]).wait()
      if prefetch:
        _gather(m + 2, rp)

    _gather(0, 0)
    _gather(1, 1)
    _body(0, 0, False, True)
    _body(1, 1, True, True)
    _body(2, 2, True, True)

    def _trip(i, _):
      j = 3 * i + 3
      for r in (0, 1, 2):
        _body(j + r, r, True, True)
      return 0
    lax.fori_loop(0, (CPT - 6) // 3, _trip, 0)
    _body(CPT - 3, 0, True, True)
    _body(CPT - 2, 1, True, False)
    _body(CPT - 1, 2, True, False)
    pltpu.make_async_copy(bufs[2], acc.at[dbufs[2]], ssems[2]).wait()
    plsc.subcore_barrier()

    pltpu.sync_copy(acc.at[pl.ds(s * ROWS_PER_TILE, ROWS_PER_TILE)],
                    out.at[c, pl.ds(s * ROWS_PER_TILE, ROWS_PER_TILE)])

  return sc_scatter


_sc_scatter_feat = _make_sc_scatter(FEAT)

_DEG_ROWS = NPAD // 128


def _make_sc_degree():
  mesh = plsc.VectorSubcoreMesh(core_axis_name="c", subcore_axis_name="s")

  @functools.partial(
      pl.kernel,
      mesh=mesh,
      out_type=jax.ShapeDtypeStruct((NC, NPAD, FEAT), jnp.float32),
      scratch_types=[
          pltpu.VMEM((CPT, CHUNK), jnp.int32),
          pltpu.VMEM((CHUNK,), jnp.int32),
          pltpu.VMEM((CHUNK,), jnp.int32),
          pltpu.VMEM((CHUNK, FEAT), jnp.float32),
          pltpu.VMEM_SHARED((NPAD, FEAT), jnp.float32),
          pltpu.SemaphoreType.DMA,
          pltpu.SemaphoreType.DMA,
      ],
  )
  def sc_degree(edges, out, pidx, didx0, didx1, onesb, acc, sems0, sems1):
    c = lax.axis_index("c")
    s = lax.axis_index("s")
    wid = c * NS + s

    def _fill(val):
      def _row(r, _):
        for k in range(FEAT // 16):
          onesb[r, pl.ds(k * 16, 16)] = jnp.full((16,), val, jnp.float32)
        return 0
      lax.fori_loop(0, CHUNK, _row, 0)

    _fill(0.0)
    for z in range(ROWS_PER_TILE // 64):
      pltpu.sync_copy(onesb.at[pl.ds(0, 64)],
                      acc.at[pl.ds(s * ROWS_PER_TILE + z * 64, 64)])
    plsc.subcore_barrier()
    _fill(1.0)

    pltpu.sync_copy(edges.at[wid], pidx)
    dbufs = (didx0, didx1)
    ssems = (sems0, sems1)

    def _unpack_dst(k, dbuf):
      for j in range(CHUNK // 16):
        pk = pidx[k, pl.ds(j * 16, 16)]
        dbuf[pl.ds(j * 16, 16)] = jnp.bitwise_and(pk, (1 << 14) - 1)

    for b in (0, 1):
      _unpack_dst(b, dbufs[b])
      pltpu.async_copy(onesb, acc.at[dbufs[b]], ssems[b], add=True)

    def _pair(i, _):
      j = 2 * i + 2
      for b in (0, 1):
        m = j + b
        pltpu.make_async_copy(onesb, acc.at[dbufs[b]], ssems[b]).wait()
        _unpack_dst(m, dbufs[b])
        pltpu.async_copy(onesb, acc.at[dbufs[b]], ssems[b], add=True)
      return 0
    lax.fori_loop(0, (CPT - 2) // 2, _pair, 0)
    for m in range(2 + 2 * ((CPT - 2) // 2), CPT):
      b = m % 2
      pltpu.make_async_copy(onesb, acc.at[dbufs[b]], ssems[b]).wait()
      _unpack_dst(m, dbufs[b])
      pltpu.async_copy(onesb, acc.at[dbufs[b]], ssems[b], add=True)
    for b in (0, 1):
      pltpu.make_async_copy(onesb, acc.at[dbufs[b]], ssems[b]).wait()
    plsc.subcore_barrier()

    pltpu.sync_copy(acc.at[pl.ds(s * ROWS_PER_TILE, ROWS_PER_TILE)],
                    out.at[c, pl.ds(s * ROWS_PER_TILE, ROWS_PER_TILE)])

  return sc_degree


_sc_degree = _make_sc_degree()


_RB = 1000
_NRB = N // _RB


def _tc_first(deg_ref, x_ref, w_ref, g_ref, dinv_ref):
  deg = deg_ref[0] + deg_ref[1] + 1.0
  dinv = lax.rsqrt(deg)
  h = jnp.dot(x_ref[...], w_ref[...], preferred_element_type=jnp.float32)
  g_ref[...] = h * dinv
  dinv_ref[...] = dinv


def _tc_mid(sp_ref, g_ref, dinv_ref, b_ref, w_ref, xo_ref, gn_ref):
  S = sp_ref[0] + sp_ref[1]
  dinv = dinv_ref[...]
  xo = jnp.maximum((S + g_ref[...]) * dinv + b_ref[...], 0.0)
  xo_ref[...] = xo
  gn_ref[...] = jnp.dot(xo, w_ref[...], preferred_element_type=jnp.float32) * dinv


def _tc_final(sp_ref, g_ref, dinv_ref, b_ref, x1_ref, x2_ref, mask_ref,
              drug_ref, l1wp_ref, l1wd_ref, l1b_ref, l2w_ref, l2b_ref,
              l3w_ref, l3b_ref, l4w_ref, l4b_ref, out_ref):
  S = sp_ref[0, :N, :] + sp_ref[1, :N, :]
  x3 = jnp.maximum((S + g_ref[...]) * dinv_ref[...] + b_ref[...], 0.0)
  xs = x1_ref[...] + x2_ref[...] + x3
  pooled = jnp.concatenate(
      [jnp.dot(mask_ref[bb:bb + 1, :], xs[P * bb:P * (bb + 1), :],
               preferred_element_type=jnp.float32) for bb in range(B)],
      axis=0)
  h = jnp.maximum(
      jnp.dot(pooled, l1wp_ref[...], preferred_element_type=jnp.float32)
      + jnp.dot(drug_ref[...], l1wd_ref[...], preferred_element_type=jnp.float32)
      + l1b_ref[...], 0.0)
  h = jnp.maximum(
      jnp.dot(h, l2w_ref[...], preferred_element_type=jnp.float32)
      + l2b_ref[...], 0.0)
  h = jnp.maximum(
      jnp.dot(h, l3w_ref[...], preferred_element_type=jnp.float32)
      + l3b_ref[...], 0.0)
  out_ref[...] = (jnp.dot(h, l4w_ref[...], preferred_element_type=jnp.float32)
                  + l4b_ref[...])


def kernel(x, protein_mask, drug_feat, W1, b1, W2, b2, W3, b3,
           L1w, L1b, L2w, L2b, L3w, L3b, L4w, L4b, edge_index):
  src = edge_index[0].astype(jnp.int32)
  dst = edge_index[1].astype(jnp.int32)
  n_pad = CPT * CHUNK - EPT
  pad_word = (jnp.left_shift(jnp.arange(n_pad, dtype=jnp.int32), 14)
              | (N + jnp.arange(n_pad, dtype=jnp.int32) % (NPAD - N)))
  edges = jnp.concatenate(
      [(jnp.left_shift(src, 14) | dst).reshape(NW, EPT),
       jnp.broadcast_to(pad_word, (NW, n_pad))], axis=1,
  ).reshape(NW, CPT, CHUNK)

  degp = _sc_degree(edges)
  degp = degp[:, :N, 0:1]

  g1, dinv = pl.pallas_call(
      _tc_first,
      grid=(_NRB,),
      in_specs=[
          pl.BlockSpec((NC, _RB, 1), lambda i: (0, i, 0)),
          pl.BlockSpec((_RB, FEAT), lambda i: (i, 0)),
          pl.BlockSpec((FEAT, FEAT), lambda i: (0, 0)),
      ],
      out_specs=[pl.BlockSpec((_RB, FEAT), lambda i: (i, 0)),
                 pl.BlockSpec((_RB, 1), lambda i: (i, 0))],
      out_shape=[jax.ShapeDtypeStruct((N, FEAT), jnp.float32),
                 jax.ShapeDtypeStruct((N, 1), jnp.float32)],
  )(degp, x, W1)

  s1 = _sc_scatter_feat(g1, edges)

  mid = pl.pallas_call(
      _tc_mid,
      grid=(_NRB,),
      in_specs=[
          pl.BlockSpec((NC, _RB, FEAT), lambda i: (0, i, 0)),
          pl.BlockSpec((_RB, FEAT), lambda i: (i, 0)),
          pl.BlockSpec((_RB, 1), lambda i: (i, 0)),
          pl.BlockSpec((1, FEAT), lambda i: (0, 0)),
          pl.BlockSpec((FEAT, FEAT), lambda i: (0, 0)),
      ],
      out_specs=[pl.BlockSpec((_RB, FEAT), lambda i: (i, 0)),
                 pl.BlockSpec((_RB, FEAT), lambda i: (i, 0))],
      out_shape=[jax.ShapeDtypeStruct((N, FEAT), jnp.float32),
                 jax.ShapeDtypeStruct((N, FEAT), jnp.float32)],
  )
  x1, g2 = mid(s1, g1, dinv, b1.reshape(1, FEAT), W2)
  s2 = _sc_scatter_feat(g2, edges)
  x2, g3 = mid(s2, g2, dinv, b2.reshape(1, FEAT), W3)
  s3 = _sc_scatter_feat(g3, edges)

  out = pl.pallas_call(
      _tc_final,
      out_shape=jax.ShapeDtypeStruct((B, 1), jnp.float32),
  )(s3, g3, dinv, b3.reshape(1, FEAT), x1, x2, protein_mask, drug_feat,
    L1w[:FEAT], L1w[FEAT:], L1b.reshape(1, -1), L2w, L2b.reshape(1, -1),
    L3w, L3b.reshape(1, -1), L4w, L4b.reshape(1, -1))
  return out


# speedup vs baseline: 1.0100x; 1.0100x over previous
import functools

import jax
import jax.numpy as jnp
from jax import lax
from jax.experimental import pallas as pl
from jax.experimental.pallas import tpu as pltpu
from jax.experimental.pallas import tpu_sc as plsc

N = 10000
NPAD = 10240
E = 320000
FEAT = 128
B = 5
P = 2000
NC = 2
NS = 16
NW = NC * NS
EPT = E // NW
CHUNK = 80
CPT = 126
PAD_DST = NPAD - 1
ROWS_PER_TILE = NPAD // NS


def _unpack_chunk(packed, k, sbuf, dbuf):
  for j in range(CHUNK // 16):
    pk = packed[k, pl.ds(j * 16, 16)]
    sbuf[pl.ds(j * 16, 16)] = lax.shift_right_logical(pk, 14)
    dbuf[pl.ds(j * 16, 16)] = jnp.bitwise_and(pk, (1 << 14) - 1)


def _make_sc_scatter(width):
  mesh = plsc.VectorSubcoreMesh(core_axis_name="c", subcore_axis_name="s")

  @functools.partial(
      pl.kernel,
      mesh=mesh,
      out_type=jax.ShapeDtypeStruct((NC, NPAD, width), jnp.float32),
      scratch_types=[
          pltpu.VMEM((CPT, CHUNK), jnp.int32),
          pltpu.VMEM((CHUNK,), jnp.int32),
          pltpu.VMEM((CHUNK,), jnp.int32),
          pltpu.VMEM((CHUNK,), jnp.int32),
          pltpu.VMEM((CHUNK,), jnp.int32),
          pltpu.VMEM((CHUNK,), jnp.int32),
          pltpu.VMEM((CHUNK,), jnp.int32),
          pltpu.VMEM((CHUNK, width), jnp.float32),
          pltpu.VMEM((CHUNK, width), jnp.float32),
          pltpu.VMEM((CHUNK, width), jnp.float32),
          pltpu.VMEM_SHARED((NPAD, width), jnp.float32),
          pltpu.SemaphoreType.DMA,
          pltpu.SemaphoreType.DMA,
          pltpu.SemaphoreType.DMA,
          pltpu.SemaphoreType.DMA,
          pltpu.SemaphoreType.DMA,
          pltpu.SemaphoreType.DMA,
      ],
  )
  def sc_scatter(table, edges, out, pidx, sidx0, sidx1, sidx2,
                 didx0, didx1, didx2, rows0, rows1, rows2, acc,
                 semg0, semg1, semg2, sems0, sems1, sems2):
    c = lax.axis_index("c")
    s = lax.axis_index("s")
    wid = c * NS + s

    def _zero_row(r, _):
      for k in range(width // 16):
        rows0[r, pl.ds(k * 16, 16)] = jnp.zeros((16,), jnp.float32)
      return 0
    lax.fori_loop(0, 64, _zero_row, 0)
    for z in range(ROWS_PER_TILE // 64):
      pltpu.sync_copy(rows0.at[pl.ds(0, 64)],
                      acc.at[pl.ds(s * ROWS_PER_TILE + z * 64, 64)])
    plsc.subcore_barrier()

    pltpu.sync_copy(edges.at[wid], pidx)

    bufs = (rows0, rows1, rows2)
    sbufs = (sidx0, sidx1, sidx2)
    dbufs = (didx0, didx1, didx2)
    gsems = (semg0, semg1, semg2)
    ssems = (sems0, sems1, sems2)

    def _gather(m, r):
      _unpack_chunk(pidx, m, sbufs[r], dbufs[r])
      pltpu.async_copy(table.at[sbufs[r]], bufs[r], gsems[r])

    def _body(m, r, wait_prev, prefetch):
      pltpu.make_async_copy(table.at[sbufs[r]], bufs[r], gsems[r]).wait()
      pltpu.async_copy(bufs[r], acc.at[dbufs[r]], ssems[r], add=True)
      rp = (r + 2) % 3
      if wait_prev:
        pltpu.make_async_copy(bufs[rp], acc.at[dbufs[rp]], ssems[rp]).wait()
      if prefetch:
        _gather(m + 2, rp)

    _gather(0, 0)
    _gather(1, 1)
    _body(0, 0, False, True)
    _body(1, 1, True, True)
    _body(2, 2, True, True)

    def _trip(i, _):
      j = 3 * i + 3
      for r in (0, 1, 2):
        _body(j + r, r, True, True)
      return 0
    lax.fori_loop(0, (CPT - 6) // 3, _trip, 0)
    _body(CPT - 3, 0, True, True)
    _body(CPT - 2, 1, True, False)
    _body(CPT - 1, 2, True, False)
    pltpu.make_async_copy(bufs[2], acc.at[dbufs[2]], ssems[2]).wait()
    plsc.subcore_barrier()

    pltpu.sync_copy(acc.at[pl.ds(s * ROWS_PER_TILE, ROWS_PER_TILE)],
                    out.at[c, pl.ds(s * ROWS_PER_TILE, ROWS_PER_TILE)])

  return sc_scatter


_sc_scatter_feat = _make_sc_scatter(FEAT)

_DEG_ROWS = NPAD // 128


def _make_sc_degree():
  mesh = plsc.VectorSubcoreMesh(core_axis_name="c", subcore_axis_name="s")

  @functools.partial(
      pl.kernel,
      mesh=mesh,
      out_type=jax.ShapeDtypeStruct((NC, NPAD, FEAT), jnp.float32),
      scratch_types=[
          pltpu.VMEM((CPT, CHUNK), jnp.int32),
          pltpu.VMEM((CHUNK,), jnp.int32),
          pltpu.VMEM((CHUNK,), jnp.int32),
          pltpu.VMEM((CHUNK, FEAT), jnp.float32),
          pltpu.VMEM_SHARED((NPAD, FEAT), jnp.float32),
          pltpu.SemaphoreType.DMA,
          pltpu.SemaphoreType.DMA,
      ],
  )
  def sc_degree(edges, out, pidx, didx0, didx1, onesb, acc, sems0, sems1):
    c = lax.axis_index("c")
    s = lax.axis_index("s")
    wid = c * NS + s

    def _fill(val):
      def _row(r, _):
        for k in range(FEAT // 16):
          onesb[r, pl.ds(k * 16, 16)] = jnp.full((16,), val, jnp.float32)
        return 0
      lax.fori_loop(0, CHUNK, _row, 0)

    _fill(0.0)
    for z in range(ROWS_PER_TILE // 64):
      pltpu.sync_copy(onesb.at[pl.ds(0, 64)],
                      acc.at[pl.ds(s * ROWS_PER_TILE + z * 64, 64)])
    plsc.subcore_barrier()
    _fill(1.0)

    pltpu.sync_copy(edges.at[wid], pidx)
    dbufs = (didx0, didx1)
    ssems = (sems0, sems1)

    def _unpack_dst(k, dbuf):
      for j in range(CHUNK // 16):
        pk = pidx[k, pl.ds(j * 16, 16)]
        dbuf[pl.ds(j * 16, 16)] = jnp.bitwise_and(pk, (1 << 14) - 1)

    for b in (0, 1):
      _unpack_dst(b, dbufs[b])
      pltpu.async_copy(onesb, acc.at[dbufs[b]], ssems[b], add=True)

    def _pair(i, _):
      j = 2 * i + 2
      for b in (0, 1):
        m = j + b
        pltpu.make_async_copy(onesb, acc.at[dbufs[b]], ssems[b]).wait()
        _unpack_dst(m, dbufs[b])
        pltpu.async_copy(onesb, acc.at[dbufs[b]], ssems[b], add=True)
      return 0
    lax.fori_loop(0, (CPT - 2) // 2, _pair, 0)
    for m in range(2 + 2 * ((CPT - 2) // 2), CPT):
      b = m % 2
      pltpu.make_async_copy(onesb, acc.at[dbufs[b]], ssems[b]).wait()
      _unpack_dst(m, dbufs[b])
      pltpu.async_copy(onesb, acc.at[dbufs[b]], ssems[b], add=True)
    for b in (0, 1):
      pltpu.make_async_copy(onesb, acc.at[dbufs[b]], ssems[b]).wait()
    plsc.subcore_barrier()

    pltpu.sync_copy(acc.at[pl.ds(s * ROWS_PER_TILE, ROWS_PER_TILE)],
                    out.at[c, pl.ds(s * ROWS_PER_TILE, ROWS_PER_TILE)])

  return sc_degree


_sc_degree = _make_sc_degree()


_RB = 1000
_NRB = N // _RB


def _tc_matmul(x_ref, w_ref, h_ref):
  h_ref[...] = jnp.dot(x_ref[...], w_ref[...],
                       preferred_element_type=jnp.float32)


def _tc_first(deg_ref, h_ref, g_ref, dinv_ref):
  deg = deg_ref[0, :, :] + deg_ref[1, :, :] + 1.0
  dinv = lax.rsqrt(deg)
  g_ref[...] = h_ref[...] * dinv
  dinv_ref[...] = dinv


def _tc_mid(sp_ref, g_ref, dinv_ref, b_ref, w_ref, xo_ref, gn_ref):
  S = sp_ref[0, :N, :] + sp_ref[1, :N, :]
  dinv = dinv_ref[...]
  xo = jnp.maximum((S + g_ref[...]) * dinv + b_ref[...], 0.0)
  xo_ref[...] = xo
  gn_ref[...] = jnp.dot(xo, w_ref[...], preferred_element_type=jnp.float32) * dinv


def _tc_final(sp_ref, g_ref, dinv_ref, b_ref, x1_ref, x2_ref, mask_ref,
              drug_ref, l1wp_ref, l1wd_ref, l1b_ref, l2w_ref, l2b_ref,
              l3w_ref, l3b_ref, l4w_ref, l4b_ref, out_ref):
  S = sp_ref[0, :N, :] + sp_ref[1, :N, :]
  x3 = jnp.maximum((S + g_ref[...]) * dinv_ref[...] + b_ref[...], 0.0)
  xs = x1_ref[...] + x2_ref[...] + x3
  pooled = jnp.concatenate(
      [jnp.dot(mask_ref[bb:bb + 1, :], xs[P * bb:P * (bb + 1), :],
               preferred_element_type=jnp.float32) for bb in range(B)],
      axis=0)
  h = jnp.maximum(
      jnp.dot(pooled, l1wp_ref[...], preferred_element_type=jnp.float32)
      + jnp.dot(drug_ref[...], l1wd_ref[...], preferred_element_type=jnp.float32)
      + l1b_ref[...], 0.0)
  h = jnp.maximum(
      jnp.dot(h, l2w_ref[...], preferred_element_type=jnp.float32)
      + l2b_ref[...], 0.0)
  h = jnp.maximum(
      jnp.dot(h, l3w_ref[...], preferred_element_type=jnp.float32)
      + l3b_ref[...], 0.0)
  out_ref[...] = (jnp.dot(h, l4w_ref[...], preferred_element_type=jnp.float32)
                  + l4b_ref[...])


def kernel(x, protein_mask, drug_feat, W1, b1, W2, b2, W3, b3,
           L1w, L1b, L2w, L2b, L3w, L3b, L4w, L4b, edge_index):
  src = edge_index[0].astype(jnp.int32)
  dst = edge_index[1].astype(jnp.int32)
  n_pad = CPT * CHUNK - EPT
  pad_word = (jnp.left_shift(jnp.arange(n_pad, dtype=jnp.int32), 14)
              | (N + jnp.arange(n_pad, dtype=jnp.int32) % (NPAD - N)))
  edges = jnp.concatenate(
      [(jnp.left_shift(src, 14) | dst).reshape(NW, EPT),
       jnp.broadcast_to(pad_word, (NW, n_pad))], axis=1,
  ).reshape(NW, CPT, CHUNK)

  degp = _sc_degree(edges)
  h1 = pl.pallas_call(
      _tc_matmul,
      out_shape=jax.ShapeDtypeStruct((N, FEAT), jnp.float32),
  )(x, W1)
  degp = degp[:, :N, 0:1]

  g1, dinv = pl.pallas_call(
      _tc_first,
      out_shape=[jax.ShapeDtypeStruct((N, FEAT), jnp.float32),
                 jax.ShapeDtypeStruct((N, 1), jnp.float32)],
  )(degp, h1)

  s1 = _sc_scatter_feat(g1, edges)

  mid = pl.pallas_call(
      _tc_mid,
      out_shape=[jax.ShapeDtypeStruct((N, FEAT), jnp.float32),
                 jax.ShapeDtypeStruct((N, FEAT), jnp.float32)],
  )
  x1, g2 = mid(s1, g1, dinv, b1.reshape(1, FEAT), W2)
  s2 = _sc_scatter_feat(g2, edges)
  x2, g3 = mid(s2, g2, dinv, b2.reshape(1, FEAT), W3)
  s3 = _sc_scatter_feat(g3, edges)

  out = pl.pallas_call(
      _tc_final,
      out_shape=jax.ShapeDtypeStruct((B, 1), jnp.float32),
  )(s3, g3, dinv, b3.reshape(1, FEAT), x1, x2, protein_mask, drug_feat,
    L1w[:FEAT], L1w[FEAT:], L1b.reshape(1, -1), L2w, L2b.reshape(1, -1),
    L3w, L3b.reshape(1, -1), L4w, L4b.reshape(1, -1))
  return out


# speedup vs baseline: 1.0142x; 1.0041x over previous
import functools

import jax
import jax.numpy as jnp
from jax import lax
from jax.experimental import pallas as pl
from jax.experimental.pallas import tpu as pltpu
from jax.experimental.pallas import tpu_sc as plsc

N = 10000
NPAD = 10240
E = 320000
FEAT = 128
B = 5
P = 2000
NC = 2
NS = 16
NW = NC * NS
EPT = E // NW
CHUNK = 80
CPT = 126
PAD_DST = NPAD - 1
ROWS_PER_TILE = NPAD // NS


def _unpack_chunk(packed, k, sbuf, dbuf):
  for j in range(CHUNK // 16):
    pk = packed[k, pl.ds(j * 16, 16)]
    sbuf[pl.ds(j * 16, 16)] = lax.shift_right_logical(pk, 14)
    dbuf[pl.ds(j * 16, 16)] = jnp.bitwise_and(pk, (1 << 14) - 1)


def _make_sc_scatter(width):
  mesh = plsc.VectorSubcoreMesh(core_axis_name="c", subcore_axis_name="s")

  @functools.partial(
      pl.kernel,
      mesh=mesh,
      out_type=jax.ShapeDtypeStruct((NC, NPAD, width), jnp.float32),
      scratch_types=[
          pltpu.VMEM((CPT, CHUNK), jnp.int32),
          pltpu.VMEM((CHUNK,), jnp.int32),
          pltpu.VMEM((CHUNK,), jnp.int32),
          pltpu.VMEM((CHUNK,), jnp.int32),
          pltpu.VMEM((CHUNK,), jnp.int32),
          pltpu.VMEM((CHUNK,), jnp.int32),
          pltpu.VMEM((CHUNK,), jnp.int32),
          pltpu.VMEM((CHUNK, width), jnp.float32),
          pltpu.VMEM((CHUNK, width), jnp.float32),
          pltpu.VMEM((CHUNK, width), jnp.float32),
          pltpu.VMEM_SHARED((NPAD, width), jnp.float32),
          pltpu.SemaphoreType.DMA,
          pltpu.SemaphoreType.DMA,
          pltpu.SemaphoreType.DMA,
          pltpu.SemaphoreType.DMA,
          pltpu.SemaphoreType.DMA,
          pltpu.SemaphoreType.DMA,
      ],
  )
  def sc_scatter(table, edges, out, pidx, sidx0, sidx1, sidx2,
                 didx0, didx1, didx2, rows0, rows1, rows2, acc,
                 semg0, semg1, semg2, sems0, sems1, sems2):
    c = lax.axis_index("c")
    s = lax.axis_index("s")
    wid = c * NS + s

    def _zero_row(r, _):
      for k in range(width // 16):
        rows0[r, pl.ds(k * 16, 16)] = jnp.zeros((16,), jnp.float32)
      return 0
    lax.fori_loop(0, 64, _zero_row, 0)
    for z in range(ROWS_PER_TILE // 64):
      pltpu.sync_copy(rows0.at[pl.ds(0, 64)],
                      acc.at[pl.ds(s * ROWS_PER_TILE + z * 64, 64)])
    plsc.subcore_barrier()

    pltpu.sync_copy(edges.at[wid], pidx)

    bufs = (rows0, rows1, rows2)
    sbufs = (sidx0, sidx1, sidx2)
    dbufs = (didx0, didx1, didx2)
    gsems = (semg0, semg1, semg2)
    ssems = (sems0, sems1, sems2)

    def _gather(m, r):
      _unpack_chunk(pidx, m, sbufs[r], dbufs[r])
      pltpu.async_copy(table.at[sbufs[r]], bufs[r], gsems[r])

    def _body(m, r, wait_prev, prefetch):
      pltpu.make_async_copy(table.at[sbufs[r]], bufs[r], gsems[r]).wait()
      pltpu.async_copy(bufs[r], acc.at[dbufs[r]], ssems[r], add=True)
      rp = (r + 2) % 3
      if wait_prev:
        pltpu.make_async_copy(bufs[rp], acc.at[dbufs[rp]], ssems[rp]).wait()
      if prefetch:
        _gather(m + 2, rp)

    _gather(0, 0)
    _gather(1, 1)
    _body(0, 0, False, True)
    _body(1, 1, True, True)
    _body(2, 2, True, True)

    def _trip(i, _):
      j = 3 * i + 3
      for r in (0, 1, 2):
        _body(j + r, r, True, True)
      return 0
    lax.fori_loop(0, (CPT - 6) // 3, _trip, 0)
    _body(CPT - 3, 0, True, True)
    _body(CPT - 2, 1, True, False)
    _body(CPT - 1, 2, True, False)
    pltpu.make_async_copy(bufs[2], acc.at[dbufs[2]], ssems[2]).wait()
    plsc.subcore_barrier()

    pltpu.sync_copy(acc.at[pl.ds(s * ROWS_PER_TILE, ROWS_PER_TILE)],
                    out.at[c, pl.ds(s * ROWS_PER_TILE, ROWS_PER_TILE)])

  return sc_scatter


_sc_scatter_feat = _make_sc_scatter(FEAT)

_DEG_ROWS = NPAD // 128


def _make_sc_degree():
  mesh = plsc.VectorSubcoreMesh(core_axis_name="c", subcore_axis_name="s")

  @functools.partial(
      pl.kernel,
      mesh=mesh,
      out_type=jax.ShapeDtypeStruct((NC, NPAD, FEAT), jnp.float32),
      scratch_types=[
          pltpu.VMEM((CPT, CHUNK), jnp.int32),
          pltpu.VMEM((CHUNK,), jnp.int32),
          pltpu.VMEM((CHUNK,), jnp.int32),
          pltpu.VMEM((CHUNK, FEAT), jnp.float32),
          pltpu.VMEM_SHARED((NPAD, FEAT), jnp.float32),
          pltpu.SemaphoreType.DMA,
          pltpu.SemaphoreType.DMA,
      ],
  )
  def sc_degree(edges, out, pidx, didx0, didx1, onesb, acc, sems0, sems1):
    c = lax.axis_index("c")
    s = lax.axis_index("s")
    wid = c * NS + s

    def _fill(val):
      def _row(r, _):
        for k in range(FEAT // 16):
          onesb[r, pl.ds(k * 16, 16)] = jnp.full((16,), val, jnp.float32)
        return 0
      lax.fori_loop(0, CHUNK, _row, 0)

    _fill(0.0)
    for z in range(ROWS_PER_TILE // 64):
      pltpu.sync_copy(onesb.at[pl.ds(0, 64)],
                      acc.at[pl.ds(s * ROWS_PER_TILE + z * 64, 64)])
    plsc.subcore_barrier()
    _fill(1.0)

    pltpu.sync_copy(edges.at[wid], pidx)
    dbufs = (didx0, didx1)
    ssems = (sems0, sems1)

    def _unpack_dst(k, dbuf):
      for j in range(CHUNK // 16):
        pk = pidx[k, pl.ds(j * 16, 16)]
        dbuf[pl.ds(j * 16, 16)] = jnp.bitwise_and(pk, (1 << 14) - 1)

    for b in (0, 1):
      _unpack_dst(b, dbufs[b])
      pltpu.async_copy(onesb, acc.at[dbufs[b]], ssems[b], add=True)

    def _pair(i, _):
      j = 2 * i + 2
      for b in (0, 1):
        m = j + b
        pltpu.make_async_copy(onesb, acc.at[dbufs[b]], ssems[b]).wait()
        _unpack_dst(m, dbufs[b])
        pltpu.async_copy(onesb, acc.at[dbufs[b]], ssems[b], add=True)
      return 0
    lax.fori_loop(0, (CPT - 2) // 2, _pair, 0)
    for m in range(2 + 2 * ((CPT - 2) // 2), CPT):
      b = m % 2
      pltpu.make_async_copy(onesb, acc.at[dbufs[b]], ssems[b]).wait()
      _unpack_dst(m, dbufs[b])
      pltpu.async_copy(onesb, acc.at[dbufs[b]], ssems[b], add=True)
    for b in (0, 1):
      pltpu.make_async_copy(onesb, acc.at[dbufs[b]], ssems[b]).wait()
    plsc.subcore_barrier()

    pltpu.sync_copy(acc.at[pl.ds(s * ROWS_PER_TILE, ROWS_PER_TILE)],
                    out.at[c, pl.ds(s * ROWS_PER_TILE, ROWS_PER_TILE)])

  return sc_degree


_sc_degree = _make_sc_degree()


_RB = 1000
_NRB = N // _RB


def _tc_first(deg_ref, x_ref, w_ref, g_ref, dinv_ref):
  deg = deg_ref[0, :, :] + deg_ref[1, :, :] + 1.0
  dinv = lax.rsqrt(deg)
  h = jnp.dot(x_ref[...], w_ref[...], preferred_element_type=jnp.float32)
  g_ref[...] = h * dinv
  dinv_ref[...] = dinv


def _tc_mid(sp_ref, g_ref, dinv_ref, b_ref, w_ref, xo_ref, gn_ref):
  S = sp_ref[0, :N, :] + sp_ref[1, :N, :]
  dinv = dinv_ref[...]
  xo = jnp.maximum((S + g_ref[...]) * dinv + b_ref[...], 0.0)
  xo_ref[...] = xo
  gn_ref[...] = jnp.dot(xo, w_ref[...], preferred_element_type=jnp.float32) * dinv


def _tc_final(sp_ref, g_ref, dinv_ref, b_ref, x1_ref, x2_ref, mask_ref,
              drug_ref, l1wp_ref, l1wd_ref, l1b_ref, l2w_ref, l2b_ref,
              l3w_ref, l3b_ref, l4w_ref, l4b_ref, out_ref):
  S = sp_ref[0, :N, :] + sp_ref[1, :N, :]
  x3 = jnp.maximum((S + g_ref[...]) * dinv_ref[...] + b_ref[...], 0.0)
  xs = x1_ref[...] + x2_ref[...] + x3
  pooled = jnp.concatenate(
      [jnp.dot(mask_ref[bb:bb + 1, :], xs[P * bb:P * (bb + 1), :],
               preferred_element_type=jnp.float32) for bb in range(B)],
      axis=0)
  h = jnp.maximum(
      jnp.dot(pooled, l1wp_ref[...], preferred_element_type=jnp.float32)
      + jnp.dot(drug_ref[...], l1wd_ref[...], preferred_element_type=jnp.float32)
      + l1b_ref[...], 0.0)
  h = jnp.maximum(
      jnp.dot(h, l2w_ref[...], preferred_element_type=jnp.float32)
      + l2b_ref[...], 0.0)
  h = jnp.maximum(
      jnp.dot(h, l3w_ref[...], preferred_element_type=jnp.float32)
      + l3b_ref[...], 0.0)
  out_ref[...] = (jnp.dot(h, l4w_ref[...], preferred_element_type=jnp.float32)
                  + l4b_ref[...])


def kernel(x, protein_mask, drug_feat, W1, b1, W2, b2, W3, b3,
           L1w, L1b, L2w, L2b, L3w, L3b, L4w, L4b, edge_index):
  src = edge_index[0].astype(jnp.int32)
  dst = edge_index[1].astype(jnp.int32)
  n_pad = CPT * CHUNK - EPT
  pad_word = (jnp.left_shift(jnp.arange(n_pad, dtype=jnp.int32), 14)
              | (N + jnp.arange(n_pad, dtype=jnp.int32) % (NPAD - N)))
  edges = jnp.concatenate(
      [(jnp.left_shift(src, 14) | dst).reshape(NW, EPT),
       jnp.broadcast_to(pad_word, (NW, n_pad))], axis=1,
  ).reshape(NW, CPT, CHUNK)

  degp = _sc_degree(edges)
  degp = degp[:, :N, 0:1]

  g1, dinv = pl.pallas_call(
      _tc_first,
      out_shape=[jax.ShapeDtypeStruct((N, FEAT), jnp.float32),
                 jax.ShapeDtypeStruct((N, 1), jnp.float32)],
  )(degp, x, W1)

  s1 = _sc_scatter_feat(g1, edges)

  mid = pl.pallas_call(
      _tc_mid,
      out_shape=[jax.ShapeDtypeStruct((N, FEAT), jnp.float32),
                 jax.ShapeDtypeStruct((N, FEAT), jnp.float32)],
  )
  x1, g2 = mid(s1, g1, dinv, b1.reshape(1, FEAT), W2)
  s2 = _sc_scatter_feat(g2, edges)
  x2, g3 = mid(s2, g2, dinv, b2.reshape(1, FEAT), W3)
  s3 = _sc_scatter_feat(g3, edges)

  out = pl.pallas_call(
      _tc_final,
      out_shape=jax.ShapeDtypeStruct((B, 1), jnp.float32),
  )(s3, g3, dinv, b3.reshape(1, FEAT), x1, x2, protein_mask, drug_feat,
    L1w[:FEAT], L1w[FEAT:], L1b.reshape(1, -1), L2w, L2b.reshape(1, -1),
    L3w, L3b.reshape(1, -1), L4w, L4b.reshape(1, -1))
  return out
